# baseline (speedup 1.0000x reference)
import functools

import jax
import jax.numpy as jnp
from jax import lax
from jax.experimental import pallas as pl
from jax.experimental.pallas import tpu as pltpu
from jax.experimental.pallas import tpu_sc as plsc

_DELTA_DIST = 1.2
_VAR_W, _DIST_W, _REG_W, _ENT_W, _LOSS_W = 1.0, 1.0, 0.001, 1.0, 0.1
_C = 20
_NW = 32
_LANES = 16
_IW = 128


def _safe_sqrt(sq):
    pos = sq > 0
    r = jnp.sqrt(jnp.where(pos, sq, 1.0))
    return jnp.where(pos, r, 0.0)


def _sc_hist_body(idx_hbm, lab_hbm, out_hbm, idx_v, lab_v, flat_v, ones_v,
                  z_v, shared, *, chunk, num_sp):
    cid = lax.axis_index("c")
    sid = lax.axis_index("s")
    wid = cid * 16 + sid
    nbins = (_C + 1) * num_sp
    region = sid * nbins

    base = wid * chunk
    pltpu.sync_copy(idx_hbm.at[pl.ds(base, chunk)], idx_v)
    pltpu.sync_copy(lab_hbm.at[pl.ds(base, chunk)], lab_v)

    zeros16 = jnp.zeros((_LANES,), jnp.float32)
    ones16 = jnp.ones((_LANES,), jnp.float32)

    def zfill(i, carry):
        z_v[pl.ds(i * _LANES, _LANES)] = zeros16
        return carry

    lax.fori_loop(0, nbins // _LANES, zfill, 0)
    pltpu.sync_copy(z_v, shared.at[pl.ds(region, nbins)])

    def ofill(i, carry):
        ones_v[pl.ds(i * _LANES, _LANES)] = ones16
        return carry

    lax.fori_loop(0, _IW // _LANES, ofill, 0)

    vregs_per_row = _IW // _LANES

    def ffill(i, carry):
        r = i // vregs_per_row
        c = lax.rem(i, vregs_per_row)
        sl = pl.ds(i * _LANES, _LANES)
        fl = lab_v[sl] * num_sp + idx_v[sl] + region
        flat_v[r, pl.ds(c * _LANES, _LANES)] = fl
        return carry

    lax.fori_loop(0, chunk // _LANES, ffill, 0)

    def scat(k, carry):
        pltpu.sync_copy(ones_v, shared.at[flat_v.at[k]], add=True)
        return carry

    lax.fori_loop(0, chunk // _IW, scat, 0)

    pltpu.sync_copy(shared.at[pl.ds(region, nbins)], out_hbm.at[wid])


def _sc_hist(idx_pad, lab_pad, num_sp):
    chunk = idx_pad.shape[0] // _NW
    nbins = (_C + 1) * num_sp
    body = functools.partial(_sc_hist_body, chunk=chunk, num_sp=num_sp)
    return pl.kernel(
        body,
        mesh=plsc.VectorSubcoreMesh(core_axis_name="c", subcore_axis_name="s"),
        out_type=jax.ShapeDtypeStruct((_NW, nbins), jnp.float32),
        scratch_types=[
            pltpu.VMEM((chunk,), jnp.int32),
            pltpu.VMEM((chunk,), jnp.int32),
            pltpu.VMEM((chunk // _IW, _IW), jnp.int32),
            pltpu.VMEM((_IW,), jnp.float32),
            pltpu.VMEM((nbins,), jnp.float32),
            pltpu.VMEM_SHARED((16 * nbins,), jnp.float32),
        ],
    )(idx_pad, lab_pad)


def _main_kernel(sp_ref, raw_ref, idx_ref, hist_ref, out_ref, spnt_bf,
                 seg_acc, *, num_blocks, num_sp):
    j = pl.program_id(0)

    @pl.when(j == 0)
    def _init():
        sp = sp_ref[...]
        n2 = jnp.sum(sp * sp, axis=1, keepdims=True)
        spn = sp / jnp.maximum(jnp.sqrt(n2), 1e-12)
        spnt_bf[...] = jnp.transpose(spn).astype(jnp.bfloat16)
        seg_acc[...] = jnp.zeros_like(seg_acc)

    raw = raw_ref[...]
    rn2 = jnp.sum(raw * raw, axis=1, keepdims=True)
    rinv = 1.0 / jnp.maximum(jnp.sqrt(rn2), 1e-12)

    idx_i = idx_ref[0]
    h = (idx_i >> 1).astype(jnp.bfloat16)
    p = (idx_i & 1).astype(jnp.bfloat16)
    one = jnp.ones_like(h)
    lhs = jnp.concatenate([h, p, one, one], axis=1)
    si = jax.lax.broadcasted_iota(jnp.int32, (1, num_sp), 1)
    rhs = jnp.concatenate(
        [2.0 * jnp.ones((1, num_sp), jnp.float32),
         jnp.ones((1, num_sp), jnp.float32),
         (-2 * (si >> 1)).astype(jnp.float32),
         (-(si & 1)).astype(jnp.float32)], axis=0).astype(jnp.bfloat16)
    diff = jnp.dot(lhs, rhs, preferred_element_type=jnp.float32)
    mask = diff == 0.0

    dmat = jnp.dot(raw.astype(jnp.bfloat16), spnt_bf[...],
                   preferred_element_type=jnp.float32)
    dot = jnp.sum(jnp.where(mask, dmat, 0.0), axis=1, keepdims=True) * rinv
    dsq = 2.0 - 2.0 * dot
    dist = _safe_sqrt(jnp.maximum(dsq, 0.0))

    seg_acc[...] += jnp.sum(jnp.where(mask, dist, 0.0), axis=0,
                            keepdims=True)

    @pl.when(j == num_blocks - 1)
    def _finish():
        sp = sp_ref[...]
        n2 = jnp.sum(sp * sp, axis=1, keepdims=True)
        spn = sp / jnp.maximum(jnp.sqrt(n2), 1e-12)

        histf = jnp.sum(hist_ref[...], axis=0)
        hist = histf[:_C, :]
        counts = jnp.sum(hist, axis=0, keepdims=True)
        seg = seg_acc[...]

        per_var = seg / jnp.maximum(counts, 1.0)
        l_var = jnp.sum(jnp.maximum(per_var, 0.0)) / num_sp

        label_sums = counts + 1e-8
        probs = hist / label_sums
        entropy = -jnp.sum(probs * jnp.log(probs + 1e-8), axis=0,
                           keepdims=True)
        valid = (label_sums > 0).astype(jnp.float32)
        l_entropy = jnp.sum(entropy * valid) / jnp.maximum(jnp.sum(valid), 1.0)

        ci = jax.lax.broadcasted_iota(jnp.int32, (_C, num_sp), 0)
        mx = jnp.max(hist, axis=0, keepdims=True)
        sel_lab = jnp.min(jnp.where(hist == mx, ci, _C), axis=0,
                          keepdims=True)
        lab_mat = (ci == jnp.broadcast_to(sel_lab, (_C, num_sp))
                   ).astype(jnp.float32)
        same_label = jax.lax.dot_general(
            lab_mat, lab_mat, (((0,), (0,)), ((), ())),
            preferred_element_type=jnp.float32)
        pair_valid = jax.lax.dot_general(
            valid, valid, (((0,), (0,)), ((), ())),
            preferred_element_type=jnp.float32)
        ri = jax.lax.broadcasted_iota(jnp.int32, (num_sp, num_sp), 0)
        cj = jax.lax.broadcasted_iota(jnp.int32, (num_sp, num_sp), 1)
        off_diag = (ri != cj).astype(jnp.float32)
        pmask = (1.0 - same_label) * off_diag * pair_valid

        gram = jax.lax.dot_general(
            spn, spn, (((1,), (1,)), ((), ())),
            preferred_element_type=jnp.float32)
        gd = jnp.sum(spn * spn, axis=1, keepdims=True)
        cdsq = gd + jnp.transpose(gd) - 2.0 * gram
        center_dist = _safe_sqrt(cdsq)
        vals = jnp.maximum(_DELTA_DIST - center_dist, 0.0) ** 2
        l_dist = jnp.sum(vals * pmask) / jnp.maximum(jnp.sum(pmask), 1.0)

        l_reg = jnp.sum(_safe_sqrt(n2)) / num_sp

        total = (_VAR_W * l_var + _DIST_W * l_dist + _REG_W * l_reg
                 + _ENT_W * l_entropy)
        out_ref[...] = jnp.reshape(total * _LOSS_W, (1, 1))


def _tc_main(superPoint_feat, rawPoint_feat, idx3, hist32, block_b):
    num_sp, feat = superPoint_feat.shape
    num_blocks = idx3.shape[0]
    body = functools.partial(_main_kernel, num_blocks=num_blocks,
                             num_sp=num_sp)
    return pl.pallas_call(
        body,
        grid=(num_blocks,),
        in_specs=[
            pl.BlockSpec((num_sp, feat), lambda j: (0, 0)),
            pl.BlockSpec((block_b, feat), lambda j: (j, 0)),
            pl.BlockSpec((1, block_b, 1), lambda j: (j, 0, 0)),
            pl.BlockSpec((_NW, _C + 1, num_sp), lambda j: (0, 0, 0)),
        ],
        out_specs=pl.BlockSpec((1, 1), lambda j: (0, 0)),
        out_shape=jax.ShapeDtypeStruct((1, 1), jnp.float32),
        scratch_shapes=[
            pltpu.VMEM((feat, num_sp), jnp.bfloat16),
            pltpu.VMEM((1, num_sp), jnp.float32),
        ],
        compiler_params=pltpu.CompilerParams(
            dimension_semantics=("arbitrary",)),
    )(superPoint_feat, rawPoint_feat, idx3, hist32)


def kernel(superPoint_feat, rawPoint_feat, raw_to_super_index, label_inds):
    num_sp = superPoint_feat.shape[0]
    n = rawPoint_feat.shape[0]

    block_b = n
    for cand in (4000, 4096, 2048, 2000, 1024, 1000, 512, 200, 100):
        if n % cand == 0:
            block_b = cand
            break
    num_blocks = n // block_b
    idx3 = raw_to_super_index.reshape(num_blocks, block_b, 1)

    chunk = -(-n // _NW)
    chunk = ((chunk + _IW - 1) // _IW) * _IW
    n_pad = chunk * _NW
    idx_pad = jnp.pad(raw_to_super_index, (0, n_pad - n))
    lab_pad = jnp.pad(label_inds, (0, n_pad - n), constant_values=_C)

    hist32 = _sc_hist(idx_pad, lab_pad, num_sp).reshape(_NW, _C + 1, num_sp)
    out = _tc_main(superPoint_feat, rawPoint_feat, idx3, hist32, block_b)
    return out[0, 0]


# speedup vs baseline: 1.4064x; 1.4064x over previous
import functools

import jax
import jax.numpy as jnp
from jax import lax
from jax.experimental import pallas as pl
from jax.experimental.pallas import tpu as pltpu
from jax.experimental.pallas import tpu_sc as plsc

_DELTA_DIST = 1.2
_VAR_W, _DIST_W, _REG_W, _ENT_W, _LOSS_W = 1.0, 1.0, 0.001, 1.0, 0.1
_C = 20
_NW = 32
_LANES = 16
_IW = 128


def _safe_sqrt(sq):
    pos = sq > 0
    r = jnp.sqrt(jnp.where(pos, sq, 1.0))
    return jnp.where(pos, r, 0.0)


def _sc_hist_body(idx_hbm, lab_hbm, out_hbm, idx_v, lab_v, flat_v, ones_v,
                  z_v, shared, *, chunk, num_sp):
    cid = lax.axis_index("c")
    sid = lax.axis_index("s")
    wid = cid * 16 + sid
    nbins = (_C + 1) * num_sp
    region = sid * nbins

    base = wid * chunk
    pltpu.sync_copy(idx_hbm.at[pl.ds(base, chunk)], idx_v)
    pltpu.sync_copy(lab_hbm.at[pl.ds(base, chunk)], lab_v)

    zeros16 = jnp.zeros((_LANES,), jnp.float32)
    ones16 = jnp.ones((_LANES,), jnp.float32)

    def zfill(i, carry):
        base8 = i * (8 * _LANES)
        for u in range(8):
            z_v[pl.ds(base8 + u * _LANES, _LANES)] = zeros16
        return carry

    lax.fori_loop(0, nbins // (8 * _LANES), zfill, 0)
    pltpu.sync_copy(z_v, shared.at[pl.ds(region, nbins)])

    def ofill(i, carry):
        ones_v[pl.ds(i * _LANES, _LANES)] = ones16
        return carry

    lax.fori_loop(0, _IW // _LANES, ofill, 0)

    vregs_per_row = _IW // _LANES

    def ffill(r, carry):
        for c in range(vregs_per_row):
            sl = pl.ds((r * vregs_per_row + c) * _LANES, _LANES)
            fl = lab_v[sl] * num_sp + idx_v[sl] + region
            flat_v[r, pl.ds(c * _LANES, _LANES)] = fl
        return carry

    lax.fori_loop(0, chunk // _IW, ffill, 0)

    def scat(k, carry):
        pltpu.sync_copy(ones_v, shared.at[flat_v.at[k]], add=True)
        return carry

    lax.fori_loop(0, chunk // _IW, scat, 0)

    pltpu.sync_copy(shared.at[pl.ds(region, nbins)], out_hbm.at[wid])


def _sc_hist(idx_pad, lab_pad, num_sp):
    chunk = idx_pad.shape[0] // _NW
    nbins = (_C + 1) * num_sp
    body = functools.partial(_sc_hist_body, chunk=chunk, num_sp=num_sp)
    return pl.kernel(
        body,
        mesh=plsc.VectorSubcoreMesh(core_axis_name="c", subcore_axis_name="s"),
        out_type=jax.ShapeDtypeStruct((_NW, nbins), jnp.float32),
        scratch_types=[
            pltpu.VMEM((chunk,), jnp.int32),
            pltpu.VMEM((chunk,), jnp.int32),
            pltpu.VMEM((chunk // _IW, _IW), jnp.int32),
            pltpu.VMEM((_IW,), jnp.float32),
            pltpu.VMEM((nbins,), jnp.float32),
            pltpu.VMEM_SHARED((16 * nbins,), jnp.float32),
        ],
    )(idx_pad, lab_pad)


def _main_kernel(sp_ref, raw_ref, idx_ref, hist_ref, out_ref, spnt_bf,
                 seg_acc, *, num_blocks, num_sp):
    j = pl.program_id(0)

    @pl.when(j == 0)
    def _init():
        sp = sp_ref[...]
        n2 = jnp.sum(sp * sp, axis=1, keepdims=True)
        spn = sp / jnp.maximum(jnp.sqrt(n2), 1e-12)
        spnt_bf[...] = jnp.transpose(spn).astype(jnp.bfloat16)
        seg_acc[...] = jnp.zeros_like(seg_acc)

    raw = raw_ref[...]
    rn2 = jnp.sum(raw * raw, axis=1, keepdims=True)
    rinv = 1.0 / jnp.maximum(jnp.sqrt(rn2), 1e-12)
    rawn_bf = (raw * rinv).astype(jnp.bfloat16)

    idx_i = idx_ref[0, 0, :][:, None]
    h = (idx_i >> 1).astype(jnp.bfloat16)
    p = (idx_i & 1).astype(jnp.bfloat16)
    one = jnp.ones_like(h)
    lhs = jnp.concatenate([h, p, one, one], axis=1)
    si = jax.lax.broadcasted_iota(jnp.int32, (1, num_sp), 1)
    rhs = jnp.concatenate(
        [2.0 * jnp.ones((1, num_sp), jnp.float32),
         jnp.ones((1, num_sp), jnp.float32),
         (-2 * (si >> 1)).astype(jnp.float32),
         (-(si & 1)).astype(jnp.float32)], axis=0).astype(jnp.bfloat16)
    diff = jnp.dot(lhs, rhs, preferred_element_type=jnp.float32)
    mask = diff == 0.0

    dmat = jnp.dot(rawn_bf, spnt_bf[...],
                   preferred_element_type=jnp.float32)
    dot = jnp.sum(jnp.where(mask, dmat, 0.0), axis=1, keepdims=True)
    dist = jnp.sqrt(jnp.maximum(2.0 - 2.0 * dot, 0.0))

    seg_acc[...] += jnp.sum(jnp.where(mask, dist, 0.0), axis=0,
                            keepdims=True)

    @pl.when(j == num_blocks - 1)
    def _finish():
        sp = sp_ref[...]
        n2 = jnp.sum(sp * sp, axis=1, keepdims=True)
        spn = sp / jnp.maximum(jnp.sqrt(n2), 1e-12)

        histf = jnp.sum(hist_ref[...], axis=0)
        hist = histf[:_C, :]
        counts = jnp.sum(hist, axis=0, keepdims=True)
        seg = seg_acc[...]

        per_var = seg / jnp.maximum(counts, 1.0)
        l_var = jnp.sum(jnp.maximum(per_var, 0.0)) / num_sp

        label_sums = counts + 1e-8
        probs = hist / label_sums
        entropy = -jnp.sum(probs * jnp.log(probs + 1e-8), axis=0,
                           keepdims=True)
        valid = (label_sums > 0).astype(jnp.float32)
        l_entropy = jnp.sum(entropy * valid) / jnp.maximum(jnp.sum(valid), 1.0)

        ci = jax.lax.broadcasted_iota(jnp.int32, (_C, num_sp), 0)
        mx = jnp.max(hist, axis=0, keepdims=True)
        sel_lab = jnp.min(jnp.where(hist == mx, ci, _C), axis=0,
                          keepdims=True)
        lab_mat = (ci == jnp.broadcast_to(sel_lab, (_C, num_sp))
                   ).astype(jnp.float32)
        same_label = jax.lax.dot_general(
            lab_mat, lab_mat, (((0,), (0,)), ((), ())),
            preferred_element_type=jnp.float32)
        pair_valid = jax.lax.dot_general(
            valid, valid, (((0,), (0,)), ((), ())),
            preferred_element_type=jnp.float32)
        ri = jax.lax.broadcasted_iota(jnp.int32, (num_sp, num_sp), 0)
        cj = jax.lax.broadcasted_iota(jnp.int32, (num_sp, num_sp), 1)
        off_diag = (ri != cj).astype(jnp.float32)
        pmask = (1.0 - same_label) * off_diag * pair_valid

        gram = jax.lax.dot_general(
            spn, spn, (((1,), (1,)), ((), ())),
            preferred_element_type=jnp.float32)
        gd = jnp.sum(spn * spn, axis=1, keepdims=True)
        cdsq = gd + jnp.transpose(gd) - 2.0 * gram
        center_dist = _safe_sqrt(cdsq)
        vals = jnp.maximum(_DELTA_DIST - center_dist, 0.0) ** 2
        l_dist = jnp.sum(vals * pmask) / jnp.maximum(jnp.sum(pmask), 1.0)

        l_reg = jnp.sum(_safe_sqrt(n2)) / num_sp

        total = (_VAR_W * l_var + _DIST_W * l_dist + _REG_W * l_reg
                 + _ENT_W * l_entropy)
        out_ref[...] = jnp.reshape(total * _LOSS_W, (1, 1))


def _tc_main(superPoint_feat, rawPoint_feat, idx3, hist32, block_b):
    num_sp, feat = superPoint_feat.shape
    num_blocks = idx3.shape[0]
    body = functools.partial(_main_kernel, num_blocks=num_blocks,
                             num_sp=num_sp)
    return pl.pallas_call(
        body,
        grid=(num_blocks,),
        in_specs=[
            pl.BlockSpec((num_sp, feat), lambda j: (0, 0)),
            pl.BlockSpec((block_b, feat), lambda j: (j, 0)),
            pl.BlockSpec((1, 1, block_b), lambda j: (j, 0, 0)),
            pl.BlockSpec((_NW, _C + 1, num_sp), lambda j: (0, 0, 0)),
        ],
        out_specs=pl.BlockSpec((1, 1), lambda j: (0, 0)),
        out_shape=jax.ShapeDtypeStruct((1, 1), jnp.float32),
        scratch_shapes=[
            pltpu.VMEM((feat, num_sp), jnp.bfloat16),
            pltpu.VMEM((1, num_sp), jnp.float32),
        ],
        compiler_params=pltpu.CompilerParams(
            dimension_semantics=("arbitrary",)),
    )(superPoint_feat, rawPoint_feat, idx3, hist32)


def kernel(superPoint_feat, rawPoint_feat, raw_to_super_index, label_inds):
    num_sp = superPoint_feat.shape[0]
    n = rawPoint_feat.shape[0]

    block_b = n
    for cand in (4000, 4096, 2048, 2000, 1024, 1000, 512, 200, 100):
        if n % cand == 0:
            block_b = cand
            break
    num_blocks = n // block_b
    idx3 = raw_to_super_index.reshape(num_blocks, 1, block_b)

    chunk = -(-n // _NW)
    chunk = ((chunk + _IW - 1) // _IW) * _IW
    n_pad = chunk * _NW
    idx_pad = jnp.pad(raw_to_super_index, (0, n_pad - n))
    lab_pad = jnp.pad(label_inds, (0, n_pad - n), constant_values=_C)

    hist32 = _sc_hist(idx_pad, lab_pad, num_sp).reshape(_NW, _C + 1, num_sp)
    out = _tc_main(superPoint_feat, rawPoint_feat, idx3, hist32, block_b)
    return out[0, 0]
